# baseline (speedup 1.0000x reference)
import functools
import math

import jax
import jax.numpy as jnp
from jax import lax
from jax.experimental import pallas as pl
from jax.experimental.pallas import tpu as pltpu
from jax.experimental.pallas import tpu_sc as plsc

VOCAB = 100000
D = 128
ROWS = 4096 * 200
NC, NS, L = 2, 16, 16
NW = NC * NS
CHUNK = 128
GPS = 2
SUP = CHUNK * GPS
B_PER_W = ROWS // NW
N_SUP = B_PER_W // SUP
N_CHUNK = B_PER_W // CHUNK
SCALE = math.sqrt(D)


def _scale_buf(rows_v, q):
    def body(r, _):
        for c in range(D // L):
            sl = pl.ds(c * L, L)
            rows_v[q, r, sl] = rows_v[q, r, sl] * SCALE
        return 0

    lax.fori_loop(0, SUP, body, 0, unroll=4)


def _emb_body(x_hbm, table_hbm, out_hbm, idx_v, rows_v, gsem, ssem):
    wid = lax.axis_index("s") * NC + lax.axis_index("c")
    base = wid * B_PER_W

    pltpu.sync_copy(x_hbm.at[wid], idx_v)

    def start_gathers(J, q):
        for g in range(GPS):
            pltpu.async_copy(table_hbm.at[idx_v.at[GPS * J + g]],
                             rows_v.at[q, pl.ds(g * CHUNK, CHUNK)], gsem.at[q])

    def wait_gathers(J, q):
        for g in range(GPS):
            pltpu.make_async_copy(table_hbm.at[idx_v.at[GPS * J + g]],
                                  rows_v.at[q, pl.ds(g * CHUNK, CHUNK)],
                                  gsem.at[q]).wait()

    def start_store(J, q):
        pltpu.async_copy(rows_v.at[q], out_hbm.at[pl.ds(base + J * SUP, SUP)],
                         ssem.at[q])

    def wait_store(J, q):
        pltpu.make_async_copy(rows_v.at[q], out_hbm.at[pl.ds(base + J * SUP, SUP)],
                              ssem.at[q]).wait()

    def super_body(J, q, first, last):
        wait_gathers(J, q)
        if not first:
            if not last:
                wait_store(J - 1, 1 - q)
                start_gathers(J + 1, 1 - q)
        else:
            start_gathers(J + 1, 1 - q)
        _scale_buf(rows_v, q)
        start_store(J, q)

    start_gathers(0, 0)

    super_body(0, 0, True, False)
    super_body(1, 1, False, False)

    def lap(gg, _):
        J0 = gg * 2
        super_body(J0, 0, False, False)
        super_body(J0 + 1, 1, False, False)
        return 0

    lax.fori_loop(1, N_SUP // 2 - 1, lap, 0)

    super_body(N_SUP - 2, 0, False, False)
    super_body(N_SUP - 1, 1, False, True)

    wait_store(N_SUP - 2, 0)
    wait_store(N_SUP - 1, 1)


_emb_call = functools.partial(
    pl.kernel,
    out_type=jax.ShapeDtypeStruct((ROWS, D), jnp.float32),
    mesh=plsc.VectorSubcoreMesh(core_axis_name="c", subcore_axis_name="s",
                                num_cores=NC, num_subcores=NS),
    scratch_types=[
        pltpu.VMEM((N_CHUNK, CHUNK), jnp.int32),
        pltpu.VMEM((2, SUP, D), jnp.float32),
        pltpu.SemaphoreType.DMA((2,)),
        pltpu.SemaphoreType.DMA((2,)),
    ],
)(_emb_body)


@jax.jit
def kernel(x, table):
    x2 = x.reshape(NW, N_CHUNK, CHUNK).astype(jnp.int32)
    out = _emb_call(x2, table)
    return out.reshape(x.shape[0], x.shape[1], D)
